# Initial kernel scaffold; baseline (speedup 1.0000x reference)
import functools

import jax
import jax.numpy as jnp
from jax.experimental import pallas as pl
from jax.experimental.pallas import tpu as pltpu

_K = 512
_ROWS_PER_BLOCK = 8
_CHUNK = 128


def _topk_mask_body(x_ref, o_ref):
    x = x_ref[...]
    r, nchunk, lanes = x.shape

    i = jax.lax.bitcast_convert_type(x, jnp.int32)
    key = i ^ ((i >> 31) & jnp.int32(0x7FFFFFFF))

    def step(b, t):
        cand = t + (jnp.int32(1) << (30 - b))
        cnt = jnp.sum((key >= cand).astype(jnp.int32), axis=(1, 2),
                      keepdims=True)
        return jnp.where(cnt >= _K, cand, t)

    t0 = jnp.full((r, 1, 1), jnp.iinfo(jnp.int32).min, jnp.int32)
    thr = jax.lax.fori_loop(0, 31, step, t0)

    gt = key > thr
    eq = key == thr
    cg = jnp.sum(gt.astype(jnp.float32), axis=(1, 2), keepdims=True)
    m = jnp.float32(_K) - cg

    eqf = eq.astype(jnp.float32)
    rows2 = jax.lax.broadcasted_iota(jnp.int32, (lanes, lanes), 0)
    cols2 = jax.lax.broadcasted_iota(jnp.int32, (lanes, lanes), 1)
    lt_incl = (rows2 <= cols2).astype(jnp.float32)
    rowsc = jax.lax.broadcasted_iota(jnp.int32, (nchunk, nchunk), 0)
    colsc = jax.lax.broadcasted_iota(jnp.int32, (nchunk, nchunk), 1)
    slt = (rowsc < colsc).astype(jnp.float32)

    within = jnp.dot(eqf.reshape(r * nchunk, lanes), lt_incl,
                     preferred_element_type=jnp.float32)
    within = within.reshape(r, nchunk, lanes)
    csum = jnp.sum(eqf, axis=2)
    prefix = jnp.dot(csum, slt, preferred_element_type=jnp.float32)
    rank = prefix[:, :, None] + within

    keep = gt | (eq & (rank <= m))
    o_ref[...] = jnp.where(keep, x, jnp.float32(0.0))


@jax.jit
def kernel(x):
    b, n = x.shape
    x3 = x.reshape(b, n // _CHUNK, _CHUNK)
    grid = (b // _ROWS_PER_BLOCK,)
    blk = (_ROWS_PER_BLOCK, n // _CHUNK, _CHUNK)
    out = pl.pallas_call(
        _topk_mask_body,
        grid=grid,
        in_specs=[pl.BlockSpec(blk, lambda ii: (ii, 0, 0))],
        out_specs=pl.BlockSpec(blk, lambda ii: (ii, 0, 0)),
        out_shape=jax.ShapeDtypeStruct(x3.shape, x3.dtype),
    )(x3)
    return out.reshape(b, n)


# speedup vs baseline: 13.3950x; 13.3950x over previous
import functools

import jax
import jax.numpy as jnp
from jax.experimental import pallas as pl
from jax.experimental.pallas import tpu as pltpu

_K = 512
_ROWS_PER_BLOCK = 8
_CHUNK = 128


def _topk_mask_body(x_ref, o_ref):
    x = x_ref[...]
    r, nchunk, lanes = x.shape

    u = jax.lax.bitcast_convert_type(x, jnp.uint32)
    neg = u >= jnp.uint32(0x80000000)
    key = jnp.where(neg, ~u, u | jnp.uint32(0x80000000))

    def step(b, t):
        cand = t | (jnp.uint32(1) << (31 - b))
        cnt = jnp.sum((key >= cand).astype(jnp.int32), axis=(1, 2),
                      keepdims=True)
        return jnp.where(cnt >= _K, cand, t)

    t0 = jnp.zeros((r, 1, 1), jnp.uint32)
    thr = jax.lax.fori_loop(0, 32, step, t0)

    gt = key > thr
    eq = key == thr
    cg = jnp.sum(gt.astype(jnp.float32), axis=(1, 2), keepdims=True)
    m = jnp.float32(_K) - cg

    eqf = eq.astype(jnp.float32)
    rows2 = jax.lax.broadcasted_iota(jnp.int32, (lanes, lanes), 0)
    cols2 = jax.lax.broadcasted_iota(jnp.int32, (lanes, lanes), 1)
    lt_incl = (rows2 <= cols2).astype(jnp.float32)
    rowsc = jax.lax.broadcasted_iota(jnp.int32, (nchunk, nchunk), 0)
    colsc = jax.lax.broadcasted_iota(jnp.int32, (nchunk, nchunk), 1)
    slt = (rowsc < colsc).astype(jnp.float32)

    within = jnp.dot(eqf.reshape(r * nchunk, lanes), lt_incl,
                     preferred_element_type=jnp.float32)
    within = within.reshape(r, nchunk, lanes)
    csum = jnp.sum(eqf, axis=2)
    prefix = jnp.dot(csum, slt, preferred_element_type=jnp.float32)
    rank = prefix[:, :, None] + within

    keep = gt | (eq & (rank <= m))
    o_ref[...] = jnp.where(keep, x, jnp.float32(0.0))


@jax.jit
def kernel(x):
    b, n = x.shape
    x3 = x.reshape(b, n // _CHUNK, _CHUNK)
    grid = (b // _ROWS_PER_BLOCK,)
    blk = (_ROWS_PER_BLOCK, n // _CHUNK, _CHUNK)
    out = pl.pallas_call(
        _topk_mask_body,
        grid=grid,
        in_specs=[pl.BlockSpec(blk, lambda ii: (ii, 0, 0))],
        out_specs=pl.BlockSpec(blk, lambda ii: (ii, 0, 0)),
        out_shape=jax.ShapeDtypeStruct(x3.shape, x3.dtype),
    )(x3)
    return out.reshape(b, n)
